# baseline (speedup 1.0000x reference)
import functools

import jax
import jax.numpy as jnp
from jax import lax
from jax.experimental import pallas as pl
from jax.experimental.pallas import tpu as pltpu
from jax.experimental.pallas import tpu_sc as plsc

B_ = 64
F_ = 8192
K_ = 4
CO_ = 16
ROWS = 256


NSEG = 128
NV = F_ // NSEG


def _topk_body(d_ref, idx_ref):
    big = jnp.int32(2 ** 30)
    inf = jnp.float32(jnp.inf)
    d3 = d_ref[...].reshape(ROWS, NV, NSEG)
    S = jnp.min(d3, axis=1)
    gcol = (lax.broadcasted_iota(jnp.int32, (ROWS, NV, NSEG), 1) * NSEG
            + lax.broadcasted_iota(jnp.int32, (ROWS, NV, NSEG), 2))
    FA = jnp.min(jnp.where(d3 == S[:, None, :], gcol, big), axis=1)

    lane_iota = lax.broadcasted_iota(jnp.int32, (ROWS, NSEG), 1)
    js = []
    Sw = S
    for t in range(K_):
        m = jnp.min(Sw, axis=1, keepdims=True)
        gm = jnp.min(jnp.where(Sw == m, FA, big), axis=1)
        jt = jnp.bitwise_and(gm, NSEG - 1)
        js.append(jt)
        if t < K_ - 1:
            Sw = jnp.where(lane_iota == jt[:, None], inf, Sw)

    J = jnp.stack(js, axis=-1)
    I = jnp.broadcast_to(J[:, None, :], (ROWS, NV, K_))
    e = jnp.take_along_axis(d3, I, axis=2, mode="promise_in_bounds")
    g = lax.broadcasted_iota(jnp.int32, (ROWS, NV, K_), 1) * NSEG + I
    e2 = e.reshape(ROWS, NV * K_)
    g2 = g.reshape(ROWS, NV * K_)
    for t in range(K_):
        m = jnp.min(e2, axis=1, keepdims=True)
        im = jnp.min(jnp.where(e2 == m, g2, big), axis=1)
        idx_ref[:, t] = im
        if t < K_ - 1:
            e2 = jnp.where(g2 == im[:, None], inf, e2)


def _topk(d2):
    return pl.pallas_call(
        _topk_body,
        grid=(F_ // ROWS,),
        in_specs=[pl.BlockSpec((ROWS, F_), lambda i: (i, 0))],
        out_specs=pl.BlockSpec((ROWS, K_), lambda i: (i, 0)),
        out_shape=jax.ShapeDtypeStruct((F_, K_), jnp.int32),
    )(d2)


def _sc_gather(x2, c2, idx_flat):
    mesh = plsc.VectorSubcoreMesh(core_axis_name="c", subcore_axis_name="s")

    @functools.partial(
        pl.kernel,
        out_type=[jax.ShapeDtypeStruct((B_, K_ * F_), jnp.float32)] * 2,
        mesh=mesh,
        scratch_types=[
            pltpu.VMEM((K_ * F_,), jnp.int32),
            pltpu.VMEM((F_,), jnp.float32),
            pltpu.VMEM((K_ * F_,), jnp.float32),
        ],
        compiler_params=pltpu.CompilerParams(needs_layout_passes=False),
    )
    def k(x_hbm, c_hbm, idx_hbm, gx_hbm, gc_hbm, idx_v, row_v, out_v):
        wid = lax.axis_index("s") * 2 + lax.axis_index("c")
        pltpu.sync_copy(idx_hbm, idx_v)
        n_chunks = (K_ * F_) // 16
        for p in range(4):
            pid = p * 32 + wid
            b = pid % B_
            src = x_hbm if p < 2 else c_hbm
            dst = gx_hbm if p < 2 else gc_hbm
            pltpu.sync_copy(src.at[b], row_v)

            def body(j, _):
                off = j * 16
                iv = idx_v[pl.ds(off, 16)]
                out_v[pl.ds(off, 16)] = plsc.load_gather(row_v, [iv])
                return 0

            lax.fori_loop(0, n_chunks, body, 0, unroll=8)
            pltpu.sync_copy(out_v, dst.at[b])

    return k(x2, c2, idx_flat)


def _conv_body(g_ref, w_ref, b_ref, o_ref):
    g = g_ref[0]
    w = w_ref[...]
    bb = b_ref[...]
    y = lax.dot_general(w, g, (((1,), (0,)), ((), ())),
                        preferred_element_type=jnp.float32)
    o_ref[0] = jnp.maximum(y + bb, 0.0)


def _conv(g, w, b2):
    return pl.pallas_call(
        _conv_body,
        grid=(B_,),
        in_specs=[
            pl.BlockSpec((1, K_, F_), lambda i: (i, 0, 0)),
            pl.BlockSpec((CO_, K_), lambda i: (0, 0)),
            pl.BlockSpec((CO_, 1), lambda i: (0, 0)),
        ],
        out_specs=pl.BlockSpec((1, CO_, F_), lambda i: (i, 0, 0)),
        out_shape=jax.ShapeDtypeStruct((B_, CO_, F_), jnp.float32),
    )(g, w, b2)


def kernel(X, Coord, distances, W, b):
    d2 = distances[0]
    idx = _topk(d2)
    idx_flat = idx.T.reshape(-1)
    x2 = X[:, 0, :]
    c2 = Coord[:, 0, :]
    gx, gc = _sc_gather(x2, c2, idx_flat)
    gx = gx.reshape(B_, K_, F_)
    gc = gc.reshape(B_, K_, F_)
    w2 = W[:, 0, :]
    b2 = b.reshape(CO_, 1)
    return (_conv(gx, w2, b2), _conv(gc, w2, b2))


# speedup vs baseline: 1.9727x; 1.9727x over previous
import functools

import jax
import jax.numpy as jnp
from jax import lax
from jax.experimental import pallas as pl
from jax.experimental.pallas import tpu as pltpu
from jax.experimental.pallas import tpu_sc as plsc

B_ = 64
F_ = 8192
K_ = 4
CO_ = 16
ROWS = 256
CONVB = 8


def _topk_body(d_ref, idx_ref):
    d = d_ref[...]
    iota = lax.broadcasted_iota(jnp.int32, (ROWS, F_), 1)
    big = jnp.int32(2 ** 30)
    inf = jnp.float32(jnp.inf)
    for t in range(K_):
        m = jnp.min(d, axis=1, keepdims=True)
        im = jnp.min(jnp.where(d == m, iota, big), axis=1)
        idx_ref[:, t] = im
        if t < K_ - 1:
            d = jnp.where(iota == im[:, None], inf, d)


def _topk(d2):
    return pl.pallas_call(
        _topk_body,
        grid=(F_ // ROWS,),
        in_specs=[pl.BlockSpec((ROWS, F_), lambda i: (i, 0))],
        out_specs=pl.BlockSpec((ROWS, K_), lambda i: (i, 0)),
        out_shape=jax.ShapeDtypeStruct((F_, K_), jnp.int32),
    )(d2)


def _sc_gather(x2, c2, idx_kf):
    mesh = plsc.VectorSubcoreMesh(core_axis_name="c", subcore_axis_name="s")

    @functools.partial(
        pl.kernel,
        out_type=[jax.ShapeDtypeStruct((B_, K_, F_), jnp.float32)] * 2,
        mesh=mesh,
        scratch_types=[
            pltpu.VMEM((K_, F_), jnp.int32),
            pltpu.VMEM((F_,), jnp.float32),
            pltpu.VMEM((K_, F_), jnp.float32),
        ],
        compiler_params=pltpu.CompilerParams(needs_layout_passes=False),
    )
    def k(x_hbm, c_hbm, idx_hbm, gx_hbm, gc_hbm, idx_v, row_v, out_v):
        wid = lax.axis_index("s") * 2 + lax.axis_index("c")
        pltpu.sync_copy(idx_hbm, idx_v)
        for p in range(4):
            pid = p * 32 + wid
            b = pid % B_
            src = x_hbm if p < 2 else c_hbm
            dst = gx_hbm if p < 2 else gc_hbm
            pltpu.sync_copy(src.at[b], row_v)

            for kk in range(K_):
                @plsc.parallel_loop(0, F_ // 128, unroll=4)
                def _(j):
                    base = j * 128
                    for c in range(8):
                        off = base + c * 16
                        iv = idx_v[kk, pl.ds(off, 16)]
                        out_v[kk, pl.ds(off, 16)] = plsc.load_gather(
                            row_v, [iv])

            pltpu.sync_copy(out_v, dst.at[b])

    return k(x2, c2, idx_kf)


def _conv_body(gx_ref, gc_ref, w_ref, b_ref, ox_ref, oc_ref):
    w = w_ref[...]
    bb = b_ref[...]
    for bi in range(CONVB):
        yx = lax.dot_general(w, gx_ref[bi], (((1,), (0,)), ((), ())),
                             preferred_element_type=jnp.float32)
        ox_ref[bi] = jnp.maximum(yx + bb, 0.0)
        yc = lax.dot_general(w, gc_ref[bi], (((1,), (0,)), ((), ())),
                             preferred_element_type=jnp.float32)
        oc_ref[bi] = jnp.maximum(yc + bb, 0.0)


def _conv(gx, gc, w, b2):
    out_sds = jax.ShapeDtypeStruct((B_, CO_, F_), jnp.float32)
    g_spec = pl.BlockSpec((CONVB, K_, F_), lambda i: (i, 0, 0))
    o_spec = pl.BlockSpec((CONVB, CO_, F_), lambda i: (i, 0, 0))
    return pl.pallas_call(
        _conv_body,
        grid=(B_ // CONVB,),
        in_specs=[
            g_spec,
            g_spec,
            pl.BlockSpec((CO_, K_), lambda i: (0, 0)),
            pl.BlockSpec((CO_, 1), lambda i: (0, 0)),
        ],
        out_specs=[o_spec, o_spec],
        out_shape=[out_sds, out_sds],
    )(gx, gc, w, b2)


def kernel(X, Coord, distances, W, b):
    d2 = distances[0]
    idx = _topk(d2)
    idx_kf = idx.T
    x2 = X[:, 0, :]
    c2 = Coord[:, 0, :]
    gx, gc = _sc_gather(x2, c2, idx_kf)
    w2 = W[:, 0, :]
    b2 = b.reshape(CO_, 1)
    ox, oc = _conv(gx, gc, w2, b2)
    return (ox, oc)
